# baseline (speedup 1.0000x reference)
import functools

import jax
import jax.numpy as jnp
from jax.experimental import pallas as pl
from jax.experimental.pallas import tpu as pltpu


def _lsh_kernel(x2_ref, rr_ref, o_ref, *, rounds, L):
    xx = x2_ref[0]
    rr = rr_ref[0]
    hT2 = jax.lax.dot_general(
        rr, xx, (((1,), (1,)), ((), ())),
        preferred_element_type=jnp.float32)
    half_tokens = xx.shape[0]
    RL = rounds * L
    iota = jax.lax.broadcasted_iota(
        jnp.int32, (L, half_tokens), 0).astype(jnp.float32)
    iota_neg = iota + float(L)
    packed = []
    for half in range(2):
        p = None
        for k in range(rounds):
            base = half * RL + k * L
            hk = hT2[base:base + L, :]
            c = jnp.abs(hk)
            w = jnp.where(hk < 0, iota_neg, iota)
            m = jnp.max(c, axis=0, keepdims=True)
            idx = jnp.where(c == m, w, float(2 * L))
            b = jnp.min(idx, axis=0, keepdims=True).astype(jnp.int32)
            p = b if p is None else p | (b << (8 * k))
        packed.append(p)
    o_ref[0] = jnp.concatenate(packed, axis=0)


def kernel(inp, rand_matrix, n_buckets):
    B, S, D = inp.shape
    _, _, R, L = rand_matrix.shape
    RL = R * L
    rt = rand_matrix.transpose(0, 2, 3, 1).reshape(B, RL, D)
    z = jnp.zeros_like(rt)
    rr = jnp.concatenate([
        jnp.concatenate([rt, z], axis=2),
        jnp.concatenate([z, rt], axis=2),
    ], axis=1)
    x2 = inp.reshape(B, S // 2, 2 * D)
    CHUNK = 4096
    grid = (B, S // CHUNK)
    packed = pl.pallas_call(
        functools.partial(_lsh_kernel, rounds=R, L=L),
        grid=grid,
        in_specs=[
            pl.BlockSpec((1, CHUNK // 2, 2 * D), lambda b, s: (b, s, 0)),
            pl.BlockSpec((1, 2 * RL, 2 * D), lambda b, s: (b, 0, 0)),
        ],
        out_specs=pl.BlockSpec((1, 2, CHUNK // 2), lambda b, s: (b, 0, s)),
        out_shape=jax.ShapeDtypeStruct((B, 2, S // 2), jnp.int32),
        compiler_params=pltpu.CompilerParams(
            dimension_semantics=("parallel", "parallel"),
        ),
    )(x2, rr)
    inter = jnp.stack([packed[:, 0, :], packed[:, 1, :]], axis=-1)
    bytes4 = jax.lax.bitcast_convert_type(inter.reshape(B, S), jnp.int8)
    return bytes4.astype(jnp.int32)


# speedup vs baseline: 2.5777x; 2.5777x over previous
import functools

import jax
import jax.numpy as jnp
from jax.experimental import pallas as pl
from jax.experimental.pallas import tpu as pltpu


def _lsh_kernel(x_ref, rt_ref, o_ref, *, rounds, L):
    xT = x_ref[0]
    rt = rt_ref[0]
    hT = jax.lax.dot_general(
        rt, xT, (((1,), (0,)), ((), ())),
        preferred_element_type=jnp.float32)
    tokens = xT.shape[1]
    iota = jax.lax.broadcasted_iota(
        jnp.int32, (L, tokens), 0).astype(jnp.float32)
    iota_neg = iota + float(L)
    parts = []
    for k in range(rounds):
        hk = hT[k * L:(k + 1) * L, :]
        c = jnp.abs(hk)
        w = jnp.where(hk < 0, iota_neg, iota)
        m = jnp.max(c, axis=0, keepdims=True)
        idx = jnp.where(c == m, w, float(2 * L))
        parts.append(jnp.min(idx, axis=0, keepdims=True).astype(jnp.int32))
    o_ref[0] = jnp.concatenate(parts, axis=0)


def kernel(inp, rand_matrix, n_buckets):
    B, S, D = inp.shape
    _, _, R, L = rand_matrix.shape
    rt = rand_matrix.transpose(0, 2, 3, 1).reshape(B, R * L, D)
    xT = inp.transpose(0, 2, 1)
    CHUNK = 4096
    grid = (B, S // CHUNK)
    out = pl.pallas_call(
        functools.partial(_lsh_kernel, rounds=R, L=L),
        grid=grid,
        in_specs=[
            pl.BlockSpec((1, D, CHUNK), lambda b, s: (b, 0, s)),
            pl.BlockSpec((1, R * L, D), lambda b, s: (b, 0, 0)),
        ],
        out_specs=pl.BlockSpec((1, R, CHUNK), lambda b, s: (b, 0, s)),
        out_shape=jax.ShapeDtypeStruct((B, R, S), jnp.int32),
        compiler_params=pltpu.CompilerParams(
            dimension_semantics=("parallel", "parallel"),
        ),
    )(xT, rt)
    return out.transpose(0, 2, 1)
